# Initial kernel scaffold; baseline (speedup 1.0000x reference)
import jax
import jax.numpy as jnp
from jax.experimental import pallas as pl


def kernel(sequence, table):
    raise NotImplementedError("write your pallas kernel here")


# speedup vs baseline: 3.9290x; 3.9290x over previous
import functools

import numpy as np
import jax
import jax.numpy as jnp
from jax import lax
from jax.experimental import pallas as pl
from jax.experimental.pallas import tpu as pltpu
from jax.experimental.pallas import tpu_sc as plsc

D = 128
V = 100000
B = 1024
L = 200
SCALE = float(np.sqrt(np.float32(D)))

NC, NS = 2, 16
NW = NC * NS
FLAT = B * L
B_PER_W = FLAT // NW
CHUNK = L
N_CHUNKS = B_PER_W // CHUNK
VPR = D // 16


def _pos_encoding(length, d):
    pos = np.arange(length)[:, np.newaxis]
    i = np.arange(d)[np.newaxis, :]
    angle_rates = 1 / np.power(10000, 2 * (i // 2) / np.float32(d))
    angle_rads = pos * angle_rates
    sines = np.sin(angle_rads[:, 0::2])
    cosines = np.cos(angle_rads[:, 1::2])
    return np.concatenate([sines, cosines], axis=-1).astype(np.float32)


_PE_NP = _pos_encoding(L, D)

_MESH = plsc.VectorSubcoreMesh(core_axis_name="c", subcore_axis_name="s")


@functools.partial(
    pl.kernel,
    out_type=jax.ShapeDtypeStruct((FLAT, D), jnp.float32),
    mesh=_MESH,
    scratch_types=[
        pltpu.VMEM((CHUNK,), jnp.int32),
        pltpu.VMEM((L, D), jnp.float32),
        pltpu.VMEM((CHUNK, D), jnp.float32),
        pltpu.SemaphoreType.DMA,
    ],
)
def _sc_embed(seq_hbm, table_hbm, pe_hbm, out_hbm, idx_v, pe_v, rows_v, sem):
    wid = lax.axis_index("s") * NC + lax.axis_index("c")
    base = wid * B_PER_W
    pltpu.sync_copy(pe_hbm, pe_v)

    def chunk_body(k, carry):
        row0 = base + k * CHUNK
        pltpu.sync_copy(seq_hbm.at[pl.ds(row0, CHUNK)], idx_v)
        pltpu.async_copy(table_hbm.at[idx_v], rows_v, sem).wait()

        def row_body(r, carry2):
            for c in range(VPR):
                sl = pl.ds(c * 16, 16)
                rows_v[r, sl] = rows_v[r, sl] * SCALE + pe_v[r, sl]
            return carry2

        lax.fori_loop(0, CHUNK, row_body, 0, unroll=False)
        pltpu.sync_copy(rows_v, out_hbm.at[pl.ds(row0, CHUNK)])
        return carry

    lax.fori_loop(0, N_CHUNKS, chunk_body, 0, unroll=False)


def kernel(sequence, table):
    seq_flat = sequence.reshape(FLAT).astype(jnp.int32)
    pe = jnp.asarray(_PE_NP)
    out = _sc_embed(seq_flat, table, pe)
    return out.reshape(B, L, D)
